# baseline (speedup 1.0000x reference)
import functools

import jax
import jax.numpy as jnp
from jax import lax
from jax.experimental import pallas as pl
from jax.experimental.pallas import tpu as pltpu
from jax.experimental.pallas import tpu_sc as plsc

N_NODES = 10000
N_EDGES = 160000
D = 128

NC = 2
NS = 16
E_PER_TEC = N_EDGES // NS
CHUNK = 80
NCHUNK = E_PER_TEC // CHUNK
RP = N_NODES // NS
ZROWS = 125
NBUF = 5


def _sc_aggregate(x, e0r, e1r):
    mesh = plsc.VectorSubcoreMesh(core_axis_name="c", subcore_axis_name="s")

    @functools.partial(
        pl.kernel,
        out_type=[
            jax.ShapeDtypeStruct((NC, N_NODES, D), jnp.float32),
            jax.ShapeDtypeStruct((NC * N_NODES, D), jnp.bfloat16),
        ],
        mesh=mesh,
        compiler_params=pltpu.CompilerParams(use_tc_tiling_on_sc=False,
                                             needs_layout_passes=False),
        scratch_types=[
            pltpu.VMEM((NCHUNK, CHUNK), jnp.int32),
            pltpu.VMEM((NCHUNK, CHUNK), jnp.int32),
        ] + [pltpu.VMEM((CHUNK, D), jnp.bfloat16)] * NBUF + [
            pltpu.VMEM((CHUNK, 16), jnp.float32),
            pltpu.VMEM((ZROWS, D), jnp.bfloat16),
            pltpu.VMEM((ZROWS, D), jnp.float32),
            pltpu.VMEM((RP, 16), jnp.float32),
            pltpu.VMEM_SHARED((N_NODES, D), jnp.bfloat16),
            pltpu.VMEM_SHARED((N_NODES, 16), jnp.float32),
        ] + [pltpu.SemaphoreType.DMA] * NBUF,
    )
    def k(x_hbm, e0_hbm, e1_hbm, agg_hbm, xh_hbm,
          src_v, dst_v, rows_a, rows_b, rows_c, rows_d, rows_e,
          ones_v, zrow_v, zrow_f, zdeg_v, agg_sh, deg_sh,
          sem_a, sem_b, sem_c, sem_d, sem_e):
        rows_ring = (rows_a, rows_b, rows_c, rows_d, rows_e)
        sem_ring = (sem_a, sem_b, sem_c, sem_d, sem_e)
        c = lax.axis_index("c")
        s = lax.axis_index("s")

        zero16 = jnp.zeros((16,), jnp.float32)
        zero32h = jnp.zeros((32,), jnp.bfloat16)
        one16 = jnp.ones((16,), jnp.float32)

        for t in range(RP // ZROWS):
            pltpu.sync_copy(x_hbm.at[pl.ds(s * RP + t * ZROWS, ZROWS)], zrow_f)

            def cvt_body(i, carry):
                for q in range(D // 32):
                    a = zrow_f[i, pl.ds(q * 32, 16)]
                    b = zrow_f[i, pl.ds(q * 32 + 16, 16)]
                    zrow_v[i, pl.ds(q * 32, 32)] = plsc.pack(
                        a, b, format=plsc.PackFormat.INTERLEAVED)
                return carry
            lax.fori_loop(0, ZROWS, cvt_body, 0)
            pltpu.sync_copy(
                zrow_v,
                xh_hbm.at[pl.ds(c * N_NODES + s * RP + t * ZROWS, ZROWS)])

        def zrow_body(i, carry):
            for j in range(D // 32):
                zrow_v[i, pl.ds(j * 32, 32)] = zero32h
            return carry
        lax.fori_loop(0, ZROWS, zrow_body, 0)

        def zdeg_body(i, carry):
            zdeg_v[i, :] = zero16
            return carry
        lax.fori_loop(0, RP, zdeg_body, 0)

        def ones_body(i, carry):
            ones_v[i, :] = one16
            return carry
        lax.fori_loop(0, CHUNK, ones_body, 0)

        for t in range(RP // ZROWS):
            pltpu.sync_copy(zrow_v, agg_sh.at[pl.ds(s * RP + t * ZROWS, ZROWS)])
        pltpu.sync_copy(zdeg_v, deg_sh.at[pl.ds(s * RP, RP)])
        plsc.subcore_barrier()

        @pl.when(c == 0)
        def _():
            pltpu.sync_copy(e0_hbm.at[0, s], src_v)
            pltpu.sync_copy(e0_hbm.at[1, s], dst_v)

        @pl.when(c == 1)
        def _():
            pltpu.sync_copy(e1_hbm.at[0, s], src_v)
            pltpu.sync_copy(e1_hbm.at[1, s], dst_v)

        cv = lax.broadcast_in_dim(c * N_NODES, (16,), ())

        def off_body(i, carry):
            for q in range(CHUNK // 16):
                src_v[i, pl.ds(q * 16, 16)] = (
                    src_v[i, pl.ds(q * 16, 16)] + cv)
            return carry
        lax.fori_loop(0, NCHUNK, off_body, 0)

        for b in range(NBUF):
            pltpu.async_copy(xh_hbm.at[src_v.at[b]], rows_ring[b], sem_ring[b])

        def ring_body(t, carry):
            for b in range(NBUF):
                j = NBUF * t + b
                pltpu.make_async_copy(
                    xh_hbm.at[src_v.at[j]], rows_ring[b], sem_ring[b]).wait()
                pltpu.sync_copy(rows_ring[b], agg_sh.at[dst_v.at[j]], add=True)

                @pl.when(j + NBUF < NCHUNK)
                def _():
                    pltpu.async_copy(
                        xh_hbm.at[src_v.at[j + NBUF]], rows_ring[b], sem_ring[b])

                pltpu.sync_copy(ones_v, deg_sh.at[dst_v.at[j]], add=True)
            return carry
        lax.fori_loop(0, NCHUNK // NBUF, ring_body, 0)
        plsc.subcore_barrier()

        pltpu.sync_copy(deg_sh.at[pl.ds(s * RP, RP)], zdeg_v)
        mask_hi = jnp.full((16,), 0xFFFF0000, jnp.uint32)
        for t in range(RP // ZROWS):
            pltpu.sync_copy(agg_sh.at[pl.ds(s * RP + t * ZROWS, ZROWS)], zrow_v)

            def norm_body(i, carry):
                dv16 = zdeg_v[t * ZROWS + i, :]
                inv16 = 1.0 / jnp.maximum(dv16, 1.0)
                for q in range(D // 32):
                    v = zrow_v[i, pl.ds(q * 32, 32)]
                    u = plsc.bitcast(v, jnp.uint32)
                    a = plsc.bitcast(u << 16, jnp.float32) * inv16
                    b = plsc.bitcast(u & mask_hi, jnp.float32) * inv16
                    zrow_f[i, pl.ds(q * 32, 16)] = a
                    zrow_f[i, pl.ds(q * 32 + 16, 16)] = b
                return carry
            lax.fori_loop(0, ZROWS, norm_body, 0)
            pltpu.sync_copy(zrow_f,
                            agg_hbm.at[c, pl.ds(s * RP + t * ZROWS, ZROWS)])

    return k(x, e0r, e1r)


def _tc_combine(agg, x, W0, W1, Wlt, b2):
    BLK = 1000
    grid = (N_NODES // BLK,)

    def body(a0_ref, a1_ref, x_ref, w0_ref, w1_ref, wlt_ref,
             b_ref, o_ref):
        a0 = a0_ref[0]
        a1 = a1_ref[0]
        o_ref[...] = (
            jnp.dot(a0, w0_ref[...], preferred_element_type=jnp.float32)
            + jnp.dot(a1, w1_ref[...], preferred_element_type=jnp.float32)
            + jnp.dot(x_ref[...], wlt_ref[...], preferred_element_type=jnp.float32)
            + b_ref[...]
        )

    return pl.pallas_call(
        body,
        grid=grid,
        in_specs=[
            pl.BlockSpec((1, BLK, D), lambda i: (0, i, 0)),
            pl.BlockSpec((1, BLK, D), lambda i: (1, i, 0)),
            pl.BlockSpec((BLK, D), lambda i: (i, 0)),
            pl.BlockSpec((D, D), lambda i: (0, 0)),
            pl.BlockSpec((D, D), lambda i: (0, 0)),
            pl.BlockSpec((D, D), lambda i: (0, 0)),
            pl.BlockSpec((1, D), lambda i: (0, 0)),
        ],
        out_specs=pl.BlockSpec((BLK, D), lambda i: (i, 0)),
        out_shape=jax.ShapeDtypeStruct((N_NODES, D), jnp.float32),
    )(agg, agg, x, W0, W1, Wlt, b2)


def kernel(x, edge_index_rel0, edge_index_rel1, W_rel0, W_rel1, W_loop, b_loop):
    e0r = edge_index_rel0.astype(jnp.int32).reshape(2, NS, NCHUNK, CHUNK)
    e1r = edge_index_rel1.astype(jnp.int32).reshape(2, NS, NCHUNK, CHUNK)
    agg, _ = _sc_aggregate(x, e0r, e1r)
    h = _tc_combine(agg, x, W_rel0, W_rel1, W_loop.T,
                    b_loop.reshape(1, D))
    return h


# speedup vs baseline: 1.0318x; 1.0318x over previous
import functools

import jax
import jax.numpy as jnp
from jax import lax
from jax.experimental import pallas as pl
from jax.experimental.pallas import tpu as pltpu
from jax.experimental.pallas import tpu_sc as plsc

N_NODES = 10000
N_EDGES = 160000
D = 128

NC = 2
NS = 16
E_PER_TEC = N_EDGES // NS
CHUNK = 80
NCHUNK = E_PER_TEC // CHUNK
RP = N_NODES // NS
ZROWS = 125
NBLK = RP // ZROWS
NBUF = 5


def _sc_aggregate(xh, e0r, e1r):
    mesh = plsc.VectorSubcoreMesh(core_axis_name="c", subcore_axis_name="s")

    @functools.partial(
        pl.kernel,
        out_type=[
            jax.ShapeDtypeStruct((NC, N_NODES, D), jnp.float32),
        ],
        mesh=mesh,
        compiler_params=pltpu.CompilerParams(use_tc_tiling_on_sc=False,
                                             needs_layout_passes=False),
        scratch_types=[
            pltpu.VMEM((NCHUNK, CHUNK), jnp.int32),
            pltpu.VMEM((NCHUNK, CHUNK), jnp.int32),
        ] + [pltpu.VMEM((CHUNK, D), jnp.bfloat16)] * NBUF + [
            pltpu.VMEM((ZROWS, D), jnp.float32),
            pltpu.VMEM((ZROWS, D), jnp.bfloat16),
            pltpu.VMEM((CHUNK, 16), jnp.float32),
            pltpu.VMEM((RP, 16), jnp.float32),
            pltpu.VMEM_SHARED((N_NODES, D), jnp.bfloat16),
            pltpu.VMEM_SHARED((N_NODES, 16), jnp.float32),
        ] + [pltpu.SemaphoreType.DMA] * NBUF,
    )
    def k(xh_hbm, e0_hbm, e1_hbm, agg_hbm,
          src_v, dst_v, rows_a, rows_b, rows_c, rows_d, rows_e,
          zf0, zv0, ones_v, zdeg_v, agg_sh, deg_sh,
          sem_a, sem_b, sem_c, sem_d, sem_e):
        rows_ring = (rows_a, rows_b, rows_c, rows_d, rows_e)
        sem_ring = (sem_a, sem_b, sem_c, sem_d, sem_e)
        c = lax.axis_index("c")
        s = lax.axis_index("s")

        zero16 = jnp.zeros((16,), jnp.float32)
        zero32h = jnp.zeros((32,), jnp.bfloat16)
        one16 = jnp.ones((16,), jnp.float32)
        mask_hi = jnp.full((16,), 0xFFFF0000, jnp.uint32)

        def aggsp(t):
            return agg_sh.at[pl.ds(s * RP + t * ZROWS, ZROWS)]

        def agghbm(t):
            return agg_hbm.at[c, pl.ds(s * RP + t * ZROWS, ZROWS)]

        def zrow_body(i, carry):
            for j in range(D // 32):
                zv0[i, pl.ds(j * 32, 32)] = zero32h
            return carry
        lax.fori_loop(0, ZROWS, zrow_body, 0)

        def zdeg_body(i, carry):
            zdeg_v[i, :] = zero16
            return carry
        lax.fori_loop(0, RP, zdeg_body, 0)

        def ones_body(i, carry):
            ones_v[i, :] = one16
            return carry
        lax.fori_loop(0, CHUNK, ones_body, 0)

        for t in range(NBLK):
            pltpu.sync_copy(zv0, aggsp(t))
        pltpu.sync_copy(zdeg_v, deg_sh.at[pl.ds(s * RP, RP)])

        @pl.when(c == 0)
        def _():
            pltpu.sync_copy(e0_hbm.at[0, s], src_v)
            pltpu.sync_copy(e0_hbm.at[1, s], dst_v)

        @pl.when(c == 1)
        def _():
            pltpu.sync_copy(e1_hbm.at[0, s], src_v)
            pltpu.sync_copy(e1_hbm.at[1, s], dst_v)

        plsc.subcore_barrier()

        for b in range(NBUF):
            pltpu.async_copy(xh_hbm.at[src_v.at[b]], rows_ring[b], sem_ring[b])

        def ring_body(t, carry):
            for b in range(NBUF):
                j = NBUF * t + b
                pltpu.make_async_copy(
                    xh_hbm.at[src_v.at[j]], rows_ring[b], sem_ring[b]).wait()
                pltpu.sync_copy(rows_ring[b], agg_sh.at[dst_v.at[j]], add=True)

                @pl.when(j + NBUF < NCHUNK)
                def _():
                    pltpu.async_copy(
                        xh_hbm.at[src_v.at[j + NBUF]], rows_ring[b], sem_ring[b])

                pltpu.sync_copy(ones_v, deg_sh.at[dst_v.at[j]], add=True)
            return carry
        lax.fori_loop(0, NCHUNK // NBUF, ring_body, 0)
        plsc.subcore_barrier()

        pltpu.sync_copy(deg_sh.at[pl.ds(s * RP, RP)], zdeg_v)
        for t in range(NBLK):
            zf, zv = zf0, zv0
            pltpu.sync_copy(aggsp(t), zv)

            def norm_body(i, carry):
                dv16 = zdeg_v[t * ZROWS + i, :]
                inv16 = 1.0 / jnp.maximum(dv16, 1.0)
                for q in range(D // 32):
                    v = zv[i, pl.ds(q * 32, 32)]
                    u = plsc.bitcast(v, jnp.uint32)
                    a = plsc.bitcast(u << 16, jnp.float32) * inv16
                    b = plsc.bitcast(u & mask_hi, jnp.float32) * inv16
                    zf[i, pl.ds(q * 32, 16)] = a
                    zf[i, pl.ds(q * 32 + 16, 16)] = b
                return carry
            lax.fori_loop(0, ZROWS, norm_body, 0)
            pltpu.sync_copy(zf, agghbm(t))

    return k(xh, e0r, e1r)


def _tc_combine(agg, x, W0, W1, Wlt, b2):
    BLK = 1000
    grid = (N_NODES // BLK,)

    def body(a0_ref, a1_ref, x_ref, w0_ref, w1_ref, wlt_ref,
             b_ref, o_ref):
        a0 = a0_ref[0]
        a1 = a1_ref[0]
        o_ref[...] = (
            jnp.dot(a0, w0_ref[...], preferred_element_type=jnp.float32)
            + jnp.dot(a1, w1_ref[...], preferred_element_type=jnp.float32)
            + jnp.dot(x_ref[...], wlt_ref[...], preferred_element_type=jnp.float32)
            + b_ref[...]
        )

    return pl.pallas_call(
        body,
        grid=grid,
        in_specs=[
            pl.BlockSpec((1, BLK, D), lambda i: (0, i, 0)),
            pl.BlockSpec((1, BLK, D), lambda i: (1, i, 0)),
            pl.BlockSpec((BLK, D), lambda i: (i, 0)),
            pl.BlockSpec((D, D), lambda i: (0, 0)),
            pl.BlockSpec((D, D), lambda i: (0, 0)),
            pl.BlockSpec((D, D), lambda i: (0, 0)),
            pl.BlockSpec((1, D), lambda i: (0, 0)),
        ],
        out_specs=pl.BlockSpec((BLK, D), lambda i: (i, 0)),
        out_shape=jax.ShapeDtypeStruct((N_NODES, D), jnp.float32),
    )(agg, agg, x, W0, W1, Wlt, b2)


_PERM = [g * 32 + (j // 2 if j % 2 == 0 else 16 + j // 2)
         for g in range(D // 32) for j in range(32)]


def kernel(x, edge_index_rel0, edge_index_rel1, W_rel0, W_rel1, W_loop, b_loop):
    e0r = edge_index_rel0.astype(jnp.int32).reshape(2, NS, NCHUNK, CHUNK)
    e1r = edge_index_rel1.astype(jnp.int32).reshape(2, NS, NCHUNK, CHUNK)
    xh = x.astype(jnp.bfloat16)[:, jnp.array(_PERM, dtype=jnp.int32)]
    agg, = _sc_aggregate(xh, e0r, e1r)
    h = _tc_combine(agg, x, W_rel0, W_rel1, W_loop.T, b_loop.reshape(1, D))
    return h


# speedup vs baseline: 1.0553x; 1.0228x over previous
import functools

import jax
import jax.numpy as jnp
from jax import lax
from jax.experimental import pallas as pl
from jax.experimental.pallas import tpu as pltpu
from jax.experimental.pallas import tpu_sc as plsc

N_NODES = 10000
N_EDGES = 160000
D = 128

NC = 2
NS = 16
E_PER_TEC = N_EDGES // NS
CHUNK = 80
NCHUNK = E_PER_TEC // CHUNK
RP = N_NODES // NS
ZROWS = 125
NBLK = RP // ZROWS
NBUF = 5


def _sc_aggregate(xh, e0r, e1r):
    mesh = plsc.VectorSubcoreMesh(core_axis_name="c", subcore_axis_name="s")

    @functools.partial(
        pl.kernel,
        out_type=[
            jax.ShapeDtypeStruct((NC, N_NODES, D), jnp.float32),
        ],
        mesh=mesh,
        compiler_params=pltpu.CompilerParams(use_tc_tiling_on_sc=False,
                                             needs_layout_passes=False),
        scratch_types=[
            pltpu.VMEM((NCHUNK, CHUNK), jnp.int32),
            pltpu.VMEM((NCHUNK, CHUNK), jnp.int32),
        ] + [pltpu.VMEM((CHUNK, D), jnp.bfloat16)] * NBUF + [
            pltpu.VMEM((ZROWS, D), jnp.float32),
            pltpu.VMEM((ZROWS, D), jnp.bfloat16),
            pltpu.VMEM((CHUNK, 16), jnp.float32),
            pltpu.VMEM((RP, 16), jnp.float32),
            pltpu.VMEM_SHARED((N_NODES, D), jnp.bfloat16),
            pltpu.VMEM_SHARED((N_NODES, 16), jnp.float32),
        ] + [pltpu.SemaphoreType.DMA] * NBUF,
    )
    def k(xh_hbm, e0_hbm, e1_hbm, agg_hbm,
          src_v, dst_v, rows_a, rows_b, rows_c, rows_d, rows_e,
          zf0, zv0, ones_v, zdeg_v, agg_sh, deg_sh,
          sem_a, sem_b, sem_c, sem_d, sem_e):
        rows_ring = (rows_a, rows_b, rows_c, rows_d, rows_e)
        sem_ring = (sem_a, sem_b, sem_c, sem_d, sem_e)
        c = lax.axis_index("c")
        s = lax.axis_index("s")

        zero16 = jnp.zeros((16,), jnp.float32)
        zero32h = jnp.zeros((32,), jnp.bfloat16)
        one16 = jnp.ones((16,), jnp.float32)
        mask_hi = jnp.full((16,), 0xFFFF0000, jnp.uint32)

        def aggsp(t):
            return agg_sh.at[pl.ds(s * RP + t * ZROWS, ZROWS)]

        def agghbm(t):
            return agg_hbm.at[c, pl.ds(s * RP + t * ZROWS, ZROWS)]

        def zrow_body(i, carry):
            for j in range(D // 32):
                zv0[i, pl.ds(j * 32, 32)] = zero32h
            return carry
        lax.fori_loop(0, ZROWS, zrow_body, 0)

        def zdeg_body(i, carry):
            zdeg_v[i, :] = zero16
            return carry
        lax.fori_loop(0, RP, zdeg_body, 0)

        def ones_body(i, carry):
            ones_v[i, :] = one16
            return carry
        lax.fori_loop(0, CHUNK, ones_body, 0)

        for t in range(NBLK):
            pltpu.sync_copy(zv0, aggsp(t))
        pltpu.sync_copy(zdeg_v, deg_sh.at[pl.ds(s * RP, RP)])

        @pl.when(c == 0)
        def _():
            pltpu.sync_copy(e0_hbm.at[0, s], src_v)
            pltpu.sync_copy(e0_hbm.at[1, s], dst_v)

        @pl.when(c == 1)
        def _():
            pltpu.sync_copy(e1_hbm.at[0, s], src_v)
            pltpu.sync_copy(e1_hbm.at[1, s], dst_v)

        plsc.subcore_barrier()

        for b in range(NBUF):
            pltpu.async_copy(xh_hbm.at[src_v.at[b]], rows_ring[b], sem_ring[b])

        def ring_body(t, carry):
            for b in range(NBUF):
                j = NBUF * t + b
                pltpu.make_async_copy(
                    xh_hbm.at[src_v.at[j]], rows_ring[b], sem_ring[b]).wait()
                pltpu.sync_copy(rows_ring[b], agg_sh.at[dst_v.at[j]], add=True)

                @pl.when(j + NBUF < NCHUNK)
                def _():
                    pltpu.async_copy(
                        xh_hbm.at[src_v.at[j + NBUF]], rows_ring[b], sem_ring[b])

                pltpu.sync_copy(ones_v, deg_sh.at[dst_v.at[j]], add=True)
            return carry
        lax.fori_loop(0, NCHUNK // NBUF, ring_body, 0)
        plsc.subcore_barrier()

        RPD = 25
        NBD = RP // RPD

        def aggspd(u):
            return agg_sh.at[pl.ds(s * RP + u * RPD, RPD)]

        def agghbmd(u):
            return agg_hbm.at[c, pl.ds(s * RP + u * RPD, RPD)]

        semi = (sem_a, sem_b)
        semo = (sem_c, sem_d)
        pltpu.sync_copy(deg_sh.at[pl.ds(s * RP, RP)], zdeg_v)
        pltpu.async_copy(aggspd(0), zv0.at[pl.ds(0, RPD)], semi[0])
        for u in range(NBD):
            pb = u % 2
            zvs = zv0.at[pl.ds(pb * RPD, RPD)]
            zfs = zf0.at[pl.ds(pb * RPD, RPD)]
            pltpu.make_async_copy(aggspd(u), zvs, semi[pb]).wait()
            if u + 1 < NBD:
                pltpu.async_copy(aggspd(u + 1),
                                 zv0.at[pl.ds(((u + 1) % 2) * RPD, RPD)],
                                 semi[(u + 1) % 2])
            if u >= 2:
                pltpu.make_async_copy(zfs, agghbmd(u - 2), semo[pb]).wait()

            def norm_body(i, carry):
                dv16 = zdeg_v[u * RPD + i, :]
                inv16 = 1.0 / jnp.maximum(dv16, 1.0)
                for q in range(D // 32):
                    v = zv0[pb * RPD + i, pl.ds(q * 32, 32)]
                    uu = plsc.bitcast(v, jnp.uint32)
                    a = plsc.bitcast(uu << 16, jnp.float32) * inv16
                    b = plsc.bitcast(uu & mask_hi, jnp.float32) * inv16
                    zf0[pb * RPD + i, pl.ds(q * 32, 16)] = a
                    zf0[pb * RPD + i, pl.ds(q * 32 + 16, 16)] = b
                return carry
            lax.fori_loop(0, RPD, norm_body, 0)
            pltpu.async_copy(zfs, agghbmd(u), semo[pb])
        pltpu.make_async_copy(zf0.at[pl.ds(((NBD - 2) % 2) * RPD, RPD)],
                              agghbmd(NBD - 2), semo[(NBD - 2) % 2]).wait()
        pltpu.make_async_copy(zf0.at[pl.ds(((NBD - 1) % 2) * RPD, RPD)],
                              agghbmd(NBD - 1), semo[(NBD - 1) % 2]).wait()

    return k(xh, e0r, e1r)


def _tc_combine(agg, x, W0, W1, Wlt, b2):
    BLK = 1000
    grid = (N_NODES // BLK,)

    def body(a0_ref, a1_ref, x_ref, w0_ref, w1_ref, wlt_ref,
             b_ref, o_ref):
        a0 = a0_ref[0]
        a1 = a1_ref[0]
        o_ref[...] = (
            jnp.dot(a0, w0_ref[...], preferred_element_type=jnp.float32)
            + jnp.dot(a1, w1_ref[...], preferred_element_type=jnp.float32)
            + jnp.dot(x_ref[...], wlt_ref[...], preferred_element_type=jnp.float32)
            + b_ref[...]
        )

    return pl.pallas_call(
        body,
        grid=grid,
        in_specs=[
            pl.BlockSpec((1, BLK, D), lambda i: (0, i, 0)),
            pl.BlockSpec((1, BLK, D), lambda i: (1, i, 0)),
            pl.BlockSpec((BLK, D), lambda i: (i, 0)),
            pl.BlockSpec((D, D), lambda i: (0, 0)),
            pl.BlockSpec((D, D), lambda i: (0, 0)),
            pl.BlockSpec((D, D), lambda i: (0, 0)),
            pl.BlockSpec((1, D), lambda i: (0, 0)),
        ],
        out_specs=pl.BlockSpec((BLK, D), lambda i: (i, 0)),
        out_shape=jax.ShapeDtypeStruct((N_NODES, D), jnp.float32),
    )(agg, agg, x, W0, W1, Wlt, b2)


_PERM = [g * 32 + (j // 2 if j % 2 == 0 else 16 + j // 2)
         for g in range(D // 32) for j in range(32)]


def kernel(x, edge_index_rel0, edge_index_rel1, W_rel0, W_rel1, W_loop, b_loop):
    e0r = edge_index_rel0.astype(jnp.int32).reshape(2, NS, NCHUNK, CHUNK)
    e1r = edge_index_rel1.astype(jnp.int32).reshape(2, NS, NCHUNK, CHUNK)
    xh = x.astype(jnp.bfloat16)[:, jnp.array(_PERM, dtype=jnp.int32)]
    agg, = _sc_aggregate(xh, e0r, e1r)
    h = _tc_combine(agg, x, W_rel0, W_rel1, W_loop.T, b_loop.reshape(1, D))
    return h
